# baseline (device time: 37402 ns/iter reference)
import jax
import jax.numpy as jnp
from jax import lax
from jax.experimental import pallas as pl
from jax.experimental.pallas import tpu as pltpu

N_Z = 4
B, S, D = 2, 256, 1024
H, Dh, Dr = 16, 64, 32
DC = 64
BS = B * S
NP = 4
HL = H // NP
CW = HL * Dh
QRW = HL * Dr
PACK = BS + 2 * CW


def _body(x_ref, wdkv_ref, wuk_ref, wuv_ref, wq_ref, wqr_ref, wkr_ref,
          wo_ref, out_ref, comm_ref, o_own, o_left, o_right, o_opp,
          zsend_sems, zrecv_sems, xsend_sems, xrecv_sems):
    my_x = lax.axis_index("x")
    my_y = lax.axis_index("y")
    my_z = lax.axis_index("z")

    r = 2 * my_x + (my_x ^ my_y)

    def ring_xy(rr):
        xx = rr // 2
        yy = lax.rem(rr, 2) ^ xx
        return xx, yy

    r_left = lax.rem(r + NP - 1, NP)
    r_right = lax.rem(r + 1, NP)
    r_opp = lax.rem(r + 2, NP)
    lx, ly = ring_xy(r_left)
    rx, ry = ring_xy(r_right)

    z_peers = tuple((my_x, my_y, lax.rem(my_z + i, N_Z)) for i in (1, 2, 3))
    xy_peers = ((rx, ry, my_z), (lx, ly, my_z), (1 - my_x, 1 - my_y, my_z))
    barrier_sem = pltpu.get_barrier_semaphore()
    for dev in z_peers + xy_peers:
        pl.semaphore_signal(
            barrier_sem, inc=1,
            device_id=dev, device_id_type=pl.DeviceIdType.MESH,
        )
    pl.semaphore_wait(barrier_sem, 6)

    bf = jnp.bfloat16
    x = jnp.concatenate([x_ref[0], x_ref[1]], axis=0).astype(bf)

    cT = lax.dot_general(
        wdkv_ref[:, :].astype(bf), x, (((0,), (1,)), ((), ())),
        preferred_element_type=jnp.float32,
    ).astype(bf)

    c0 = r * CW
    q0 = r * QRW
    wuk_c = wuk_ref[:, pl.ds(c0, CW)].astype(bf)
    wuv_c = wuv_ref[:, pl.ds(c0, CW)].astype(bf)

    comm_ref[0, :, :BS] = cT
    comm_ref[0, :, BS:BS + CW] = wuk_c
    comm_ref[0, :, BS + CW:] = wuv_c

    def contract(chunk):
        cT_j = chunk[:, :BS]
        wuk_j = chunk[:, BS:BS + CW]
        wuv_j = chunk[:, BS + CW:]
        k = lax.dot_general(cT_j, wuk_j, (((0,), (0,)), ((), ())),
                            preferred_element_type=jnp.float32)
        v = lax.dot_general(cT_j, wuv_j, (((0,), (0,)), ((), ())),
                            preferred_element_type=jnp.float32)
        return k, v

    zrdmas = []
    for i in (1, 2, 3):
        rdma = pltpu.make_async_remote_copy(
            src_ref=comm_ref.at[0],
            dst_ref=comm_ref.at[N_Z - i],
            send_sem=zsend_sems.at[i - 1],
            recv_sem=zrecv_sems.at[N_Z - i - 1],
            device_id=(my_x, my_y, lax.rem(my_z + i, N_Z)),
            device_id_type=pl.DeviceIdType.MESH,
        )
        rdma.start()
        zrdmas.append(rdma)

    K = lax.dot_general(cT, wuk_c, (((0,), (0,)), ((), ())),
                        preferred_element_type=jnp.float32)
    V = lax.dot_general(cT, wuv_c, (((0,), (0,)), ((), ())),
                        preferred_element_type=jnp.float32)
    Qc = jnp.dot(x, wq_ref[:, pl.ds(c0, CW)].astype(bf),
                 preferred_element_type=jnp.float32).astype(bf)
    Qr = jnp.dot(x, wqr_ref[:, pl.ds(q0, QRW)].astype(bf),
                 preferred_element_type=jnp.float32).astype(bf)
    Kr = jnp.dot(x, wkr_ref[:, :].astype(bf),
                 preferred_element_type=jnp.float32).astype(bf)

    for j in (1, 2, 3):
        recv = pltpu.make_async_remote_copy(
            src_ref=comm_ref.at[0],
            dst_ref=comm_ref.at[j],
            send_sem=zsend_sems.at[0],
            recv_sem=zrecv_sems.at[j - 1],
            device_id=(my_x, my_y, my_z),
            device_id_type=pl.DeviceIdType.MESH,
        )
        recv.wait_recv()
        k_j, v_j = contract(comm_ref[j])
        K = K + k_j
        V = V + v_j

    K = K.astype(bf)
    V = V.astype(bf)
    scale = (Dh + Dr) ** -0.5

    xy_devs = ((rx, ry, my_z), (lx, ly, my_z), (1 - my_x, 1 - my_y, my_z))
    xy_dsts = (o_left, o_right, o_opp)

    def xcopy(dst_buf, hp, pi, dev):
        return pltpu.make_async_remote_copy(
            src_ref=o_own.at[:, hp * 2 * Dh:(hp + 1) * 2 * Dh],
            dst_ref=dst_buf.at[:, hp * 2 * Dh:(hp + 1) * 2 * Dh],
            send_sem=xsend_sems.at[hp * 3 + pi],
            recv_sem=xrecv_sems.at[hp * 3 + pi],
            device_id=dev, device_id_type=pl.DeviceIdType.MESH,
        )

    xrdmas = []
    for hh in range(HL):
        h0 = hh * Dh
        for b in range(B):
            r0 = b * S
            kr_b = Kr[r0:r0 + S, :]
            q = Qc[r0:r0 + S, h0:h0 + Dh]
            k = K[r0:r0 + S, h0:h0 + Dh]
            v = V[r0:r0 + S, h0:h0 + Dh]
            qr = Qr[r0:r0 + S, hh * Dr:(hh + 1) * Dr]
            s_qk = lax.dot_general(q, k, (((1,), (1,)), ((), ())),
                                   preferred_element_type=jnp.float32)
            s_r = lax.dot_general(qr, kr_b, (((1,), (1,)), ((), ())),
                                  preferred_element_type=jnp.float32)
            scores = (s_qk + s_r) * scale
            m = jnp.max(scores, axis=-1, keepdims=True)
            p = jnp.exp(scores - m)
            p = (p / jnp.sum(p, axis=-1, keepdims=True)).astype(bf)
            o_own[r0:r0 + S, h0:h0 + Dh] = jnp.dot(
                p, v, preferred_element_type=jnp.float32).astype(bf)
        if hh % 2 == 1:
            for pi, dev in enumerate(xy_devs):
                rdma = xcopy(xy_dsts[pi], hh // 2, pi, dev)
                rdma.start()
                xrdmas.append(rdma)

    def proj(o_blk, rb):
        return lax.dot_general(
            o_blk, wo_ref[pl.ds(rb * CW, CW), :].astype(bf),
            (((1,), (0,)), ((), ())),
            preferred_element_type=jnp.float32)

    out_acc = proj(o_own[:, :], r)

    for pi, (o_buf, rb) in enumerate(
            ((o_left, r_left), (o_right, r_right), (o_opp, r_opp))):
        for hp in range(HL // 2):
            xcopy(o_buf, hp, pi, xy_devs[pi]).wait_recv()
        out_acc = out_acc + proj(o_buf[:, :], rb)

    out_ref[0] = out_acc[:S]
    out_ref[1] = out_acc[S:]

    for rdma in zrdmas + xrdmas:
        rdma.wait_send()


def kernel(x, Wdkv, Wuk, Wuv, Wq, Wqr, Wkr, Wo):
    return pl.pallas_call(
        _body,
        out_shape=jax.ShapeDtypeStruct((B, S, D), jnp.float32),
        in_specs=[pl.BlockSpec(memory_space=pltpu.VMEM)] * 8,
        out_specs=pl.BlockSpec(memory_space=pltpu.VMEM),
        scratch_shapes=[
            pltpu.VMEM((N_Z, DC, PACK), jnp.bfloat16),
            pltpu.VMEM((BS, CW), jnp.bfloat16),
            pltpu.VMEM((BS, CW), jnp.bfloat16),
            pltpu.VMEM((BS, CW), jnp.bfloat16),
            pltpu.VMEM((BS, CW), jnp.bfloat16),
            pltpu.SemaphoreType.DMA((N_Z - 1,)),
            pltpu.SemaphoreType.DMA((N_Z - 1,)),
            pltpu.SemaphoreType.DMA((HL // 2 * 3,)),
            pltpu.SemaphoreType.DMA((HL // 2 * 3,)),
        ],
        compiler_params=pltpu.CompilerParams(collective_id=0),
    )(x, Wdkv, Wuk, Wuv, Wq, Wqr, Wkr, Wo)


# device time: 35328 ns/iter; 1.0587x vs baseline; 1.0587x over previous
import jax
import jax.numpy as jnp
from jax import lax
from jax.experimental import pallas as pl
from jax.experimental.pallas import tpu as pltpu

N_Z = 4
B, S, D = 2, 256, 1024
H, Dh, Dr = 16, 64, 32
DC = 64
BS = B * S
NP = 4
HL = H // NP
CW = HL * Dh
QRW = HL * Dr
PACK = BS + 2 * CW


def _body(x_ref, wdkv_ref, wuk_ref, wuv_ref, wq_ref, wqr_ref, wkr_ref,
          wo_ref, out_ref, comm_ref, o_own, o_left, o_right, o_opp,
          zsend_sems, zrecv_sems, xsend_sems, xrecv_sems):
    my_x = lax.axis_index("x")
    my_y = lax.axis_index("y")
    my_z = lax.axis_index("z")

    r = 2 * my_x + (my_x ^ my_y)

    def ring_xy(rr):
        xx = rr // 2
        yy = lax.rem(rr, 2) ^ xx
        return xx, yy

    r_left = lax.rem(r + NP - 1, NP)
    r_right = lax.rem(r + 1, NP)
    r_opp = lax.rem(r + 2, NP)
    lx, ly = ring_xy(r_left)
    rx, ry = ring_xy(r_right)

    z_peers = tuple((my_x, my_y, lax.rem(my_z + i, N_Z)) for i in (1, 2, 3))
    xy_peers = ((rx, ry, my_z), (lx, ly, my_z), (1 - my_x, 1 - my_y, my_z))
    barrier_sem = pltpu.get_barrier_semaphore()
    for dev in z_peers + xy_peers:
        pl.semaphore_signal(
            barrier_sem, inc=1,
            device_id=dev, device_id_type=pl.DeviceIdType.MESH,
        )
    pl.semaphore_wait(barrier_sem, 6)

    bf = jnp.bfloat16
    x = jnp.concatenate([x_ref[0], x_ref[1]], axis=0)

    cT = lax.dot_general(
        wdkv_ref[:, :], x, (((0,), (1,)), ((), ())),
        preferred_element_type=jnp.float32,
    ).astype(bf)

    c0 = r * CW
    q0 = r * QRW
    wuk_c = wuk_ref[:, pl.ds(c0, CW)]
    wuv_c = wuv_ref[:, pl.ds(c0, CW)]

    comm_ref[0, :, :BS] = cT
    comm_ref[0, :, BS:BS + CW] = wuk_c
    comm_ref[0, :, BS + CW:] = wuv_c

    def contract(chunk):
        cT_j = chunk[:, :BS]
        wuk_j = chunk[:, BS:BS + CW]
        wuv_j = chunk[:, BS + CW:]
        k = lax.dot_general(cT_j, wuk_j, (((0,), (0,)), ((), ())),
                            preferred_element_type=jnp.float32)
        v = lax.dot_general(cT_j, wuv_j, (((0,), (0,)), ((), ())),
                            preferred_element_type=jnp.float32)
        return k, v

    zrdmas = []
    for i in (1, 2, 3):
        rdma = pltpu.make_async_remote_copy(
            src_ref=comm_ref.at[0],
            dst_ref=comm_ref.at[N_Z - i],
            send_sem=zsend_sems.at[i - 1],
            recv_sem=zrecv_sems.at[N_Z - i - 1],
            device_id=(my_x, my_y, lax.rem(my_z + i, N_Z)),
            device_id_type=pl.DeviceIdType.MESH,
        )
        rdma.start()
        zrdmas.append(rdma)

    K = lax.dot_general(cT, wuk_c, (((0,), (0,)), ((), ())),
                        preferred_element_type=jnp.float32)
    V = lax.dot_general(cT, wuv_c, (((0,), (0,)), ((), ())),
                        preferred_element_type=jnp.float32)
    Qc = jnp.dot(x, wq_ref[:, pl.ds(c0, CW)],
                 preferred_element_type=jnp.float32).astype(bf)
    Qr = jnp.dot(x, wqr_ref[:, pl.ds(q0, QRW)],
                 preferred_element_type=jnp.float32).astype(bf)
    Kr = jnp.dot(x, wkr_ref[:, :],
                 preferred_element_type=jnp.float32).astype(bf)

    for j in (1, 2, 3):
        recv = pltpu.make_async_remote_copy(
            src_ref=comm_ref.at[0],
            dst_ref=comm_ref.at[j],
            send_sem=zsend_sems.at[0],
            recv_sem=zrecv_sems.at[j - 1],
            device_id=(my_x, my_y, my_z),
            device_id_type=pl.DeviceIdType.MESH,
        )
        recv.wait_recv()
        k_j, v_j = contract(comm_ref[j])
        K = K + k_j
        V = V + v_j

    K = K.astype(bf)
    V = V.astype(bf)
    scale = (Dh + Dr) ** -0.5

    xy_devs = ((rx, ry, my_z), (lx, ly, my_z), (1 - my_x, 1 - my_y, my_z))
    xy_dsts = (o_left, o_right, o_opp)

    def xcopy(dst_buf, hp, pi, dev):
        return pltpu.make_async_remote_copy(
            src_ref=o_own.at[:, hp * 2 * Dh:(hp + 1) * 2 * Dh],
            dst_ref=dst_buf.at[:, hp * 2 * Dh:(hp + 1) * 2 * Dh],
            send_sem=xsend_sems.at[hp * 3 + pi],
            recv_sem=xrecv_sems.at[hp * 3 + pi],
            device_id=dev, device_id_type=pl.DeviceIdType.MESH,
        )

    xrdmas = []
    for hh in range(HL):
        h0 = hh * Dh
        for b in range(B):
            r0 = b * S
            kr_b = Kr[r0:r0 + S, :]
            q = Qc[r0:r0 + S, h0:h0 + Dh]
            k = K[r0:r0 + S, h0:h0 + Dh]
            v = V[r0:r0 + S, h0:h0 + Dh]
            qr = Qr[r0:r0 + S, hh * Dr:(hh + 1) * Dr]
            s_qk = lax.dot_general(q, k, (((1,), (1,)), ((), ())),
                                   preferred_element_type=jnp.float32)
            s_r = lax.dot_general(qr, kr_b, (((1,), (1,)), ((), ())),
                                  preferred_element_type=jnp.float32)
            scores = (s_qk + s_r) * scale
            m = jnp.max(scores, axis=-1, keepdims=True)
            p = jnp.exp(scores - m)
            p = (p / jnp.sum(p, axis=-1, keepdims=True)).astype(bf)
            o_own[r0:r0 + S, h0:h0 + Dh] = jnp.dot(
                p, v, preferred_element_type=jnp.float32).astype(bf)
        if hh % 2 == 1:
            for pi, dev in enumerate(xy_devs):
                rdma = xcopy(xy_dsts[pi], hh // 2, pi, dev)
                rdma.start()
                xrdmas.append(rdma)

    def proj(o_blk, rb):
        return lax.dot_general(
            o_blk, wo_ref[pl.ds(rb * CW, CW), :],
            (((1,), (0,)), ((), ())),
            preferred_element_type=jnp.float32)

    out_acc = proj(o_own[:, :], r)

    for pi, (o_buf, rb) in enumerate(
            ((o_left, r_left), (o_right, r_right), (o_opp, r_opp))):
        for hp in range(HL // 2):
            xcopy(o_buf, hp, pi, xy_devs[pi]).wait_recv()
        out_acc = out_acc + proj(o_buf[:, :], rb)

    out_ref[0] = out_acc[:S]
    out_ref[1] = out_acc[S:]

    for rdma in zrdmas + xrdmas:
        rdma.wait_send()


def kernel(x, Wdkv, Wuk, Wuv, Wq, Wqr, Wkr, Wo):
    args = [a.astype(jnp.bfloat16)
            for a in (x, Wdkv, Wuk, Wuv, Wq, Wqr, Wkr, Wo)]
    return pl.pallas_call(
        _body,
        out_shape=jax.ShapeDtypeStruct((B, S, D), jnp.float32),
        in_specs=[pl.BlockSpec(memory_space=pltpu.VMEM)] * 8,
        out_specs=pl.BlockSpec(memory_space=pltpu.VMEM),
        scratch_shapes=[
            pltpu.VMEM((N_Z, DC, PACK), jnp.bfloat16),
            pltpu.VMEM((BS, CW), jnp.bfloat16),
            pltpu.VMEM((BS, CW), jnp.bfloat16),
            pltpu.VMEM((BS, CW), jnp.bfloat16),
            pltpu.VMEM((BS, CW), jnp.bfloat16),
            pltpu.SemaphoreType.DMA((N_Z - 1,)),
            pltpu.SemaphoreType.DMA((N_Z - 1,)),
            pltpu.SemaphoreType.DMA((HL // 2 * 3,)),
            pltpu.SemaphoreType.DMA((HL // 2 * 3,)),
        ],
        compiler_params=pltpu.CompilerParams(collective_id=0),
    )(*args)


# device time: 32999 ns/iter; 1.1334x vs baseline; 1.0706x over previous
import jax
import jax.numpy as jnp
from jax import lax
from jax.experimental import pallas as pl
from jax.experimental.pallas import tpu as pltpu

N_Z = 4
B, S, D = 2, 256, 1024
H, Dh, Dr = 16, 64, 32
DC = 64
BS = B * S
NP = 4
HL = H // NP
CW = HL * Dh
QRW = HL * Dr
PACK = BS + 2 * CW


def _body(x_ref, wdkv_ref, wuk_ref, wuv_ref, wq_ref, wqr_ref, wkr_ref,
          wo_ref, out_ref, comm_ref, o_own, o_left, o_right, o_opp,
          zsend_sems, zrecv_sems, xsend_sems, xrecv_sems):
    my_x = lax.axis_index("x")
    my_y = lax.axis_index("y")
    my_z = lax.axis_index("z")

    r = 2 * my_x + (my_x ^ my_y)

    def ring_xy(rr):
        xx = rr // 2
        yy = lax.rem(rr, 2) ^ xx
        return xx, yy

    r_left = lax.rem(r + NP - 1, NP)
    r_right = lax.rem(r + 1, NP)
    r_opp = lax.rem(r + 2, NP)
    lx, ly = ring_xy(r_left)
    rx, ry = ring_xy(r_right)

    z_peers = tuple((my_x, my_y, lax.rem(my_z + i, N_Z)) for i in (1, 2, 3))
    xy_peers = ((rx, ry, my_z), (lx, ly, my_z), (1 - my_x, 1 - my_y, my_z))
    barrier_sem = pltpu.get_barrier_semaphore()
    for dev in z_peers + xy_peers:
        pl.semaphore_signal(
            barrier_sem, inc=1,
            device_id=dev, device_id_type=pl.DeviceIdType.MESH,
        )

    bf = jnp.bfloat16
    x = jnp.concatenate([x_ref[0], x_ref[1]], axis=0)

    cT = lax.dot_general(
        wdkv_ref[:, :], x, (((0,), (1,)), ((), ())),
        preferred_element_type=jnp.float32,
    ).astype(bf)

    c0 = r * CW
    q0 = r * QRW
    wuk_c = wuk_ref[:, pl.ds(c0, CW)]
    wuv_c = wuv_ref[:, pl.ds(c0, CW)]

    comm_ref[0, :, :BS] = cT
    comm_ref[0, :, BS:BS + CW] = wuk_c
    comm_ref[0, :, BS + CW:] = wuv_c

    def contract(chunk):
        cT_j = chunk[:, :BS]
        wuk_j = chunk[:, BS:BS + CW]
        wuv_j = chunk[:, BS + CW:]
        k = lax.dot_general(cT_j, wuk_j, (((0,), (0,)), ((), ())),
                            preferred_element_type=jnp.float32)
        v = lax.dot_general(cT_j, wuv_j, (((0,), (0,)), ((), ())),
                            preferred_element_type=jnp.float32)
        return k, v

    pl.semaphore_wait(barrier_sem, 6)

    zrdmas = []
    for i in (1, 2, 3):
        rdma = pltpu.make_async_remote_copy(
            src_ref=comm_ref.at[0],
            dst_ref=comm_ref.at[N_Z - i],
            send_sem=zsend_sems.at[i - 1],
            recv_sem=zrecv_sems.at[N_Z - i - 1],
            device_id=(my_x, my_y, lax.rem(my_z + i, N_Z)),
            device_id_type=pl.DeviceIdType.MESH,
        )
        rdma.start()
        zrdmas.append(rdma)

    K = lax.dot_general(cT, wuk_c, (((0,), (0,)), ((), ())),
                        preferred_element_type=jnp.float32)
    V = lax.dot_general(cT, wuv_c, (((0,), (0,)), ((), ())),
                        preferred_element_type=jnp.float32)
    Qc = jnp.dot(x, wq_ref[:, pl.ds(c0, CW)],
                 preferred_element_type=jnp.float32).astype(bf)
    Qr = jnp.dot(x, wqr_ref[:, pl.ds(q0, QRW)],
                 preferred_element_type=jnp.float32).astype(bf)
    Kr = jnp.dot(x, wkr_ref[:, :],
                 preferred_element_type=jnp.float32).astype(bf)
    s_r_pre = {}
    for _b in range(B):
        _r0 = _b * S
        _kr = Kr[_r0:_r0 + S, :]
        for _hh in range(HL):
            _qr = Qr[_r0:_r0 + S, _hh * Dr:(_hh + 1) * Dr]
            s_r_pre[(_b, _hh)] = lax.dot_general(
                _qr, _kr, (((1,), (1,)), ((), ())),
                preferred_element_type=jnp.float32)

    for j in (1, 2, 3):
        recv = pltpu.make_async_remote_copy(
            src_ref=comm_ref.at[0],
            dst_ref=comm_ref.at[j],
            send_sem=zsend_sems.at[0],
            recv_sem=zrecv_sems.at[j - 1],
            device_id=(my_x, my_y, my_z),
            device_id_type=pl.DeviceIdType.MESH,
        )
        recv.wait_recv()
        k_j, v_j = contract(comm_ref[j])
        K = K + k_j
        V = V + v_j

    K = K.astype(bf)
    V = V.astype(bf)
    scale = (Dh + Dr) ** -0.5

    xy_devs = ((rx, ry, my_z), (lx, ly, my_z), (1 - my_x, 1 - my_y, my_z))
    xy_dsts = (o_left, o_right, o_opp)

    def xcopy(dst_buf, hp, pi, dev):
        return pltpu.make_async_remote_copy(
            src_ref=o_own.at[:, hp * 2 * Dh:(hp + 1) * 2 * Dh],
            dst_ref=dst_buf.at[:, hp * 2 * Dh:(hp + 1) * 2 * Dh],
            send_sem=xsend_sems.at[hp * 3 + pi],
            recv_sem=xrecv_sems.at[hp * 3 + pi],
            device_id=dev, device_id_type=pl.DeviceIdType.MESH,
        )

    xrdmas = []
    for hh in range(HL):
        h0 = hh * Dh
        for b in range(B):
            r0 = b * S
            kr_b = Kr[r0:r0 + S, :]
            q = Qc[r0:r0 + S, h0:h0 + Dh]
            k = K[r0:r0 + S, h0:h0 + Dh]
            v = V[r0:r0 + S, h0:h0 + Dh]
            s_qk = lax.dot_general(q, k, (((1,), (1,)), ((), ())),
                                   preferred_element_type=jnp.float32)
            scores = (s_qk + s_r_pre[(b, hh)]) * scale
            m = jnp.max(scores, axis=-1, keepdims=True)
            p = jnp.exp(scores - m)
            p = (p / jnp.sum(p, axis=-1, keepdims=True)).astype(bf)
            o_own[r0:r0 + S, h0:h0 + Dh] = jnp.dot(
                p, v, preferred_element_type=jnp.float32).astype(bf)
        if hh % 2 == 1:
            for pi, dev in enumerate(xy_devs):
                rdma = xcopy(xy_dsts[pi], hh // 2, pi, dev)
                rdma.start()
                xrdmas.append(rdma)

    def proj(o_blk, rb):
        return lax.dot_general(
            o_blk, wo_ref[pl.ds(rb * CW, CW), :],
            (((1,), (0,)), ((), ())),
            preferred_element_type=jnp.float32)

    out_acc = proj(o_own[:, :], r)

    for pi, (o_buf, rb) in enumerate(
            ((o_left, r_left), (o_right, r_right), (o_opp, r_opp))):
        for hp in range(HL // 2):
            xcopy(o_buf, hp, pi, xy_devs[pi]).wait_recv()
        out_acc = out_acc + proj(o_buf[:, :], rb)

    out_ref[0] = out_acc[:S]
    out_ref[1] = out_acc[S:]

    for rdma in zrdmas + xrdmas:
        rdma.wait_send()


def kernel(x, Wdkv, Wuk, Wuv, Wq, Wqr, Wkr, Wo):
    args = [a.astype(jnp.bfloat16)
            for a in (x, Wdkv, Wuk, Wuv, Wq, Wqr, Wkr, Wo)]
    return pl.pallas_call(
        _body,
        out_shape=jax.ShapeDtypeStruct((B, S, D), jnp.float32),
        in_specs=[pl.BlockSpec(memory_space=pltpu.VMEM)] * 8,
        out_specs=pl.BlockSpec(memory_space=pltpu.VMEM),
        scratch_shapes=[
            pltpu.VMEM((N_Z, DC, PACK), jnp.bfloat16),
            pltpu.VMEM((BS, CW), jnp.bfloat16),
            pltpu.VMEM((BS, CW), jnp.bfloat16),
            pltpu.VMEM((BS, CW), jnp.bfloat16),
            pltpu.VMEM((BS, CW), jnp.bfloat16),
            pltpu.SemaphoreType.DMA((N_Z - 1,)),
            pltpu.SemaphoreType.DMA((N_Z - 1,)),
            pltpu.SemaphoreType.DMA((HL // 2 * 3,)),
            pltpu.SemaphoreType.DMA((HL // 2 * 3,)),
        ],
        compiler_params=pltpu.CompilerParams(collective_id=0),
    )(*args)


# device time: 31057 ns/iter; 1.2043x vs baseline; 1.0625x over previous
import jax
import jax.numpy as jnp
from jax import lax
from jax.experimental import pallas as pl
from jax.experimental.pallas import tpu as pltpu

N_Z = 4
B, S, D = 2, 256, 1024
H, Dh, Dr = 16, 64, 32
DC = 64
BS = B * S
NP = 4
HL = H // NP
CW = HL * Dh
QRW = HL * Dr
PACK = BS + 2 * CW


def _body(x_ref, wdkv_ref, wuk_ref, wuv_ref, wq_ref, wqr_ref, wkr_ref,
          wo_ref, out_ref, comm_ref, o_own, o_left, o_right, o_opp,
          zsend_sems, zrecv_sems, xsend_sems, xrecv_sems):
    my_x = lax.axis_index("x")
    my_y = lax.axis_index("y")
    my_z = lax.axis_index("z")

    r = 2 * my_x + (my_x ^ my_y)

    def ring_xy(rr):
        xx = rr // 2
        yy = lax.rem(rr, 2) ^ xx
        return xx, yy

    r_left = lax.rem(r + NP - 1, NP)
    r_right = lax.rem(r + 1, NP)
    r_opp = lax.rem(r + 2, NP)
    lx, ly = ring_xy(r_left)
    rx, ry = ring_xy(r_right)

    z_peers = tuple((my_x, my_y, lax.rem(my_z + i, N_Z)) for i in (1, 2, 3))
    xy_peers = ((rx, ry, my_z), (lx, ly, my_z), (1 - my_x, 1 - my_y, my_z))
    barrier_sem = pltpu.get_barrier_semaphore()
    for dev in z_peers + xy_peers:
        pl.semaphore_signal(
            barrier_sem, inc=1,
            device_id=dev, device_id_type=pl.DeviceIdType.MESH,
        )

    bf = jnp.bfloat16
    x = jnp.concatenate([x_ref[0], x_ref[1]], axis=0)

    cT = lax.dot_general(
        wdkv_ref[:, :], x, (((0,), (1,)), ((), ())),
        preferred_element_type=jnp.float32,
    ).astype(bf)

    wuk_c = wuk_ref[:, :]
    wuv_c = wuv_ref[:, :]

    comm_ref[0, :, :BS] = cT
    comm_ref[0, :, BS:BS + CW] = wuk_c
    comm_ref[0, :, BS + CW:] = wuv_c

    def contract(chunk):
        cT_j = chunk[:, :BS]
        wuk_j = chunk[:, BS:BS + CW]
        wuv_j = chunk[:, BS + CW:]
        k = lax.dot_general(cT_j, wuk_j, (((0,), (0,)), ((), ())),
                            preferred_element_type=jnp.float32)
        v = lax.dot_general(cT_j, wuv_j, (((0,), (0,)), ((), ())),
                            preferred_element_type=jnp.float32)
        return k, v

    pl.semaphore_wait(barrier_sem, 6)

    zrdmas = []
    for i in (1, 2, 3):
        rdma = pltpu.make_async_remote_copy(
            src_ref=comm_ref.at[0],
            dst_ref=comm_ref.at[N_Z - i],
            send_sem=zsend_sems.at[i - 1],
            recv_sem=zrecv_sems.at[N_Z - i - 1],
            device_id=(my_x, my_y, lax.rem(my_z + i, N_Z)),
            device_id_type=pl.DeviceIdType.MESH,
        )
        rdma.start()
        zrdmas.append(rdma)

    K = lax.dot_general(cT, wuk_c, (((0,), (0,)), ((), ())),
                        preferred_element_type=jnp.float32)
    V = lax.dot_general(cT, wuv_c, (((0,), (0,)), ((), ())),
                        preferred_element_type=jnp.float32)
    Qc = jnp.dot(x, wq_ref[:, :],
                 preferred_element_type=jnp.float32).astype(bf)
    Qr = jnp.dot(x, wqr_ref[:, :],
                 preferred_element_type=jnp.float32).astype(bf)
    Kr = jnp.dot(x, wkr_ref[:, :],
                 preferred_element_type=jnp.float32).astype(bf)
    s_r_pre = {}
    for _b in range(B):
        _r0 = _b * S
        _kr = Kr[_r0:_r0 + S, :]
        for _hh in range(HL):
            _qr = Qr[_r0:_r0 + S, _hh * Dr:(_hh + 1) * Dr]
            s_r_pre[(_b, _hh)] = lax.dot_general(
                _qr, _kr, (((1,), (1,)), ((), ())),
                preferred_element_type=jnp.float32)

    for j in (1, 2, 3):
        recv = pltpu.make_async_remote_copy(
            src_ref=comm_ref.at[0],
            dst_ref=comm_ref.at[j],
            send_sem=zsend_sems.at[0],
            recv_sem=zrecv_sems.at[j - 1],
            device_id=(my_x, my_y, my_z),
            device_id_type=pl.DeviceIdType.MESH,
        )
        recv.wait_recv()
        k_j, v_j = contract(comm_ref[j])
        K = K + k_j
        V = V + v_j

    K = K.astype(bf)
    V = V.astype(bf)
    scale = (Dh + Dr) ** -0.5

    xy_devs = ((rx, ry, my_z), (lx, ly, my_z), (1 - my_x, 1 - my_y, my_z))
    xy_dsts = (o_left, o_right, o_opp)

    def xcopy(dst_buf, hp, pi, dev):
        return pltpu.make_async_remote_copy(
            src_ref=o_own.at[:, hp * 2 * Dh:(hp + 1) * 2 * Dh],
            dst_ref=dst_buf.at[:, hp * 2 * Dh:(hp + 1) * 2 * Dh],
            send_sem=xsend_sems.at[hp * 3 + pi],
            recv_sem=xrecv_sems.at[hp * 3 + pi],
            device_id=dev, device_id_type=pl.DeviceIdType.MESH,
        )

    xrdmas = []
    for hh in range(HL):
        h0 = hh * Dh
        for b in range(B):
            r0 = b * S
            kr_b = Kr[r0:r0 + S, :]
            q = Qc[r0:r0 + S, h0:h0 + Dh]
            k = K[r0:r0 + S, h0:h0 + Dh]
            v = V[r0:r0 + S, h0:h0 + Dh]
            s_qk = lax.dot_general(q, k, (((1,), (1,)), ((), ())),
                                   preferred_element_type=jnp.float32)
            scores = (s_qk + s_r_pre[(b, hh)]) * scale
            m = jnp.max(scores, axis=-1, keepdims=True)
            p = jnp.exp(scores - m)
            p = (p / jnp.sum(p, axis=-1, keepdims=True)).astype(bf)
            o_own[r0:r0 + S, h0:h0 + Dh] = jnp.dot(
                p, v, preferred_element_type=jnp.float32).astype(bf)
        if hh % 2 == 1:
            for pi, dev in enumerate(xy_devs):
                rdma = xcopy(xy_dsts[pi], hh // 2, pi, dev)
                rdma.start()
                xrdmas.append(rdma)

    def proj(o_blk, rb):
        return lax.dot_general(
            o_blk, wo_ref[pl.ds(rb * CW, CW), :],
            (((1,), (0,)), ((), ())),
            preferred_element_type=jnp.float32)

    out_acc = proj(o_own[:, :], r)

    for pi, (o_buf, rb) in enumerate(
            ((o_left, r_left), (o_right, r_right), (o_opp, r_opp))):
        for hp in range(HL // 2):
            xcopy(o_buf, hp, pi, xy_devs[pi]).wait_recv()
        out_acc = out_acc + proj(o_buf[:, :], rb)

    out_ref[0] = out_acc[:S]
    out_ref[1] = out_acc[S:]

    for rdma in zrdmas + xrdmas:
        rdma.wait_send()


def kernel(x, Wdkv, Wuk, Wuv, Wq, Wqr, Wkr, Wo):
    mx = lax.axis_index("x")
    my = lax.axis_index("y")
    rr = 2 * mx + (mx ^ my)
    Wuk = lax.dynamic_slice_in_dim(Wuk, rr * CW, CW, 1)
    Wuv = lax.dynamic_slice_in_dim(Wuv, rr * CW, CW, 1)
    Wq = lax.dynamic_slice_in_dim(Wq, rr * CW, CW, 1)
    Wqr = lax.dynamic_slice_in_dim(Wqr, rr * QRW, QRW, 1)
    args = [a.astype(jnp.bfloat16)
            for a in (x, Wdkv, Wuk, Wuv, Wq, Wqr, Wkr, Wo)]
    return pl.pallas_call(
        _body,
        out_shape=jax.ShapeDtypeStruct((B, S, D), jnp.float32),
        in_specs=[pl.BlockSpec(memory_space=pltpu.VMEM)] * 8,
        out_specs=pl.BlockSpec(memory_space=pltpu.VMEM),
        scratch_shapes=[
            pltpu.VMEM((N_Z, DC, PACK), jnp.bfloat16),
            pltpu.VMEM((BS, CW), jnp.bfloat16),
            pltpu.VMEM((BS, CW), jnp.bfloat16),
            pltpu.VMEM((BS, CW), jnp.bfloat16),
            pltpu.VMEM((BS, CW), jnp.bfloat16),
            pltpu.SemaphoreType.DMA((N_Z - 1,)),
            pltpu.SemaphoreType.DMA((N_Z - 1,)),
            pltpu.SemaphoreType.DMA((HL // 2 * 3,)),
            pltpu.SemaphoreType.DMA((HL // 2 * 3,)),
        ],
        compiler_params=pltpu.CompilerParams(collective_id=0),
    )(*args)
